# Initial kernel scaffold; baseline (speedup 1.0000x reference)
import jax
import jax.numpy as jnp
from jax.experimental import pallas as pl


def kernel(x, edge_index, batch, W_enc, b_enc, W0, b0, g0, be0, W1, b1, g1, be1, W2, b2, g2, be2, Wm1, bm1, Wm2, bm2, Wm3, bm3):
    raise NotImplementedError("write your pallas kernel here")


# speedup vs baseline: 8.4967x; 8.4967x over previous
import functools

import jax
import jax.numpy as jnp
import numpy as np
from jax import lax
from jax.experimental import pallas as pl
from jax.experimental.pallas import tpu as pltpu
from jax.experimental.pallas import tpu_sc as plsc

N = 10000
E = 320000
HID = 128
NG = 64
FFN = 300
EPS = 1e-5

NC = 2
NS = 16
NW = NC * NS
C = 128
CPW = 79
E_PAD = NW * CPW * C
RPT = 632
N_PAD = RPT * NS

DW = 16
BN = 1000
GRID = N // BN


def _sc_degree_body(dst_hbm, ones_hbm, zeros_hbm, out_hbm, idx_d, ones_v, acc):
    cid = lax.axis_index("c")
    sid = lax.axis_index("s")
    w = sid * NC + cid
    pltpu.sync_copy(zeros_hbm.at[pl.ds(sid * RPT, RPT)],
                    acc.at[pl.ds(sid * RPT, RPT)])
    pltpu.sync_copy(ones_hbm, ones_v)
    plsc.subcore_barrier()

    def body(ci, carry):
        base = (w * CPW + ci) * C
        pltpu.sync_copy(dst_hbm.at[pl.ds(base, C)], idx_d)
        pltpu.sync_copy(ones_v, acc.at[idx_d], add=True)
        return carry

    lax.fori_loop(0, CPW, body, 0)
    plsc.subcore_barrier()
    pltpu.sync_copy(acc.at[pl.ds(sid * RPT, RPT)],
                    out_hbm.at[cid, pl.ds(sid * RPT, RPT)])


def _sc_spmm_body(src_hbm, dst_hbm, mt_hbm, zeros_hbm, out_hbm,
                  idx_s, idx_d, rows, acc, sem):
    cid = lax.axis_index("c")
    sid = lax.axis_index("s")
    w = sid * NC + cid
    pltpu.sync_copy(zeros_hbm.at[pl.ds(sid * RPT, RPT)],
                    acc.at[pl.ds(sid * RPT, RPT)])
    plsc.subcore_barrier()

    def body(ci, carry):
        base = (w * CPW + ci) * C
        pltpu.sync_copy(src_hbm.at[pl.ds(base, C)], idx_s)
        pltpu.async_copy(mt_hbm.at[idx_s], rows, sem).wait()
        pltpu.sync_copy(dst_hbm.at[pl.ds(base, C)], idx_d)
        pltpu.sync_copy(rows, acc.at[idx_d], add=True)
        return carry

    lax.fori_loop(0, CPW, body, 0)
    plsc.subcore_barrier()
    pltpu.sync_copy(acc.at[pl.ds(sid * RPT, RPT)],
                    out_hbm.at[cid, pl.ds(sid * RPT, RPT)])


@functools.lru_cache(maxsize=None)
def _sc_kernels():
    mesh = plsc.VectorSubcoreMesh(
        core_axis_name="c", subcore_axis_name="s",
        num_cores=NC, num_subcores=NS)
    sc_degree = pl.kernel(
        _sc_degree_body,
        out_type=jax.ShapeDtypeStruct((NC, N_PAD, HID), jnp.float32),
        mesh=mesh,
        scratch_types=[
            pltpu.VMEM((C,), jnp.int32),
            pltpu.VMEM((C, HID), jnp.float32),
            pltpu.VMEM_SHARED((N_PAD, HID), jnp.float32),
        ],
    )
    sc_spmm = pl.kernel(
        _sc_spmm_body,
        out_type=jax.ShapeDtypeStruct((NC, N_PAD, HID), jnp.float32),
        mesh=mesh,
        scratch_types=[
            pltpu.VMEM((C,), jnp.int32),
            pltpu.VMEM((C,), jnp.int32),
            pltpu.VMEM((C, HID), jnp.float32),
            pltpu.VMEM_SHARED((N_PAD, HID), jnp.float32),
            pltpu.SemaphoreType.DMA,
        ],
    )
    return sc_degree, sc_spmm


_ISQ = float(1.0 / np.sqrt(np.float32(1.0) + np.float32(EPS)))


def _enc_body(deg_ref, x_ref, wencT_ref, benc_ref, w0T_ref, mt0_ref, dis_ref):
    deg = deg_ref[0][:, :1] + deg_ref[1][:, :1] + 1.0
    dis = lax.rsqrt(deg)
    h = jnp.dot(x_ref[...], wencT_ref[...],
                preferred_element_type=jnp.float32) + benc_ref[...]
    mt0_ref[...] = jnp.dot(h, w0T_ref[...],
                           preferred_element_type=jnp.float32) * dis
    dis_ref[...] = dis


def _layer_body(has_res, acc_ref, mt_ref, dis_ref, hprev_ref, b_ref, g_ref,
                be_ref, wnT_ref, h_ref, mtn_ref):
    dis = dis_ref[...]
    agg = (acc_ref[0] + acc_ref[1] + mt_ref[...]) * dis
    h2 = (agg + b_ref[...]) * _ISQ * g_ref[...] + be_ref[...]
    h = jnp.maximum(h2, 0.0)
    if has_res:
        h = h + hprev_ref[...]
    h_ref[...] = h
    mtn_ref[...] = jnp.dot(h, wnT_ref[...],
                           preferred_element_type=jnp.float32) * dis


def _final_body(acc_ref, mt_ref, dis_ref, hprev_ref, b_ref, g_ref, be_ref,
                batch_ref, sums_ref, cnt_ref):
    i = pl.program_id(0)
    agg = (acc_ref[0] + acc_ref[1] + mt_ref[...]) * dis_ref[...]
    h2 = (agg + b_ref[...]) * _ISQ * g_ref[...] + be_ref[...]
    h = jnp.maximum(h2, 0.0) + hprev_ref[...]
    onehot = (batch_ref[...] ==
              lax.broadcasted_iota(jnp.int32, (1, NG), 1)).astype(jnp.float32)
    ps = lax.dot_general(onehot, h, (((0,), (0,)), ((), ())),
                         preferred_element_type=jnp.float32)
    pc = lax.dot_general(onehot, jnp.ones((BN, 1), jnp.float32),
                         (((0,), (0,)), ((), ())),
                         preferred_element_type=jnp.float32)

    @pl.when(i == 0)
    def _():
        sums_ref[...] = ps
        cnt_ref[...] = pc

    @pl.when(i > 0)
    def _():
        sums_ref[...] += ps
        cnt_ref[...] += pc


def _mlp_body(sums_ref, cnt_ref, w1T_ref, b1_ref, w2T_ref, b2_ref, w3T_ref,
              b3_ref, o_ref):
    gm = sums_ref[...] / jnp.maximum(cnt_ref[...], 1.0)
    o = jnp.maximum(jnp.dot(gm, w1T_ref[...],
                            preferred_element_type=jnp.float32) + b1_ref[...],
                    0.0)
    o = jnp.maximum(jnp.dot(o, w2T_ref[...],
                            preferred_element_type=jnp.float32) + b2_ref[...],
                    0.0)
    o_ref[...] = jnp.dot(o, w3T_ref[...],
                         preferred_element_type=jnp.float32) + b3_ref[...]


def _row_spec(cols):
    return pl.BlockSpec((BN, cols), lambda i: (i, 0))


_ACC_SPEC_1 = pl.BlockSpec((NC, BN, HID), lambda i: (0, i, 0))
_ACC_SPEC_H = pl.BlockSpec((NC, BN, HID), lambda i: (0, i, 0))
_FULL = pl.BlockSpec(index_map=lambda i: (0, 0))


def _tc_enc(deg2, x, wencT, benc, w0T):
    return pl.pallas_call(
        _enc_body,
        grid=(GRID,),
        in_specs=[_ACC_SPEC_1, _row_spec(HID), _FULL, _FULL, _FULL],
        out_specs=[_row_spec(HID), _row_spec(1)],
        out_shape=[jax.ShapeDtypeStruct((N, HID), jnp.float32),
                   jax.ShapeDtypeStruct((N, 1), jnp.float32)],
    )(deg2, x, wencT, benc, w0T)


def _tc_layer(acc2, mt, dis, hprev, b, g, be, wnT, has_res):
    return pl.pallas_call(
        functools.partial(_layer_body, has_res),
        grid=(GRID,),
        in_specs=[_ACC_SPEC_H, _row_spec(HID), _row_spec(1), _row_spec(HID),
                  _FULL, _FULL, _FULL, _FULL],
        out_specs=[_row_spec(HID), _row_spec(HID)],
        out_shape=[jax.ShapeDtypeStruct((N, HID), jnp.float32),
                   jax.ShapeDtypeStruct((N, HID), jnp.float32)],
    )(acc2, mt, dis, hprev, b, g, be, wnT)


def _tc_final(acc2, mt, dis, hprev, b, g, be, batch2d):
    return pl.pallas_call(
        _final_body,
        grid=(GRID,),
        in_specs=[_ACC_SPEC_H, _row_spec(HID), _row_spec(1), _row_spec(HID),
                  _FULL, _FULL, _FULL, _row_spec(1)],
        out_specs=[pl.BlockSpec((NG, HID), lambda i: (0, 0)),
                   pl.BlockSpec((NG, 1), lambda i: (0, 0))],
        out_shape=[jax.ShapeDtypeStruct((NG, HID), jnp.float32),
                   jax.ShapeDtypeStruct((NG, 1), jnp.float32)],
    )(acc2, mt, dis, hprev, b, g, be, batch2d)


def _tc_mlp(sums, cnt, w1T, b1, w2T, b2, w3T, b3):
    return pl.pallas_call(
        _mlp_body,
        out_shape=jax.ShapeDtypeStruct((NG, 1), jnp.float32),
    )(sums, cnt, w1T, b1, w2T, b2, w3T, b3)


def kernel(x, edge_index, batch, W_enc, b_enc, W0, b0, g0, be0, W1, b1, g1,
           be1, W2, b2, g2, be2, Wm1, bm1, Wm2, bm2, Wm3, bm3):
    src = edge_index[0]
    dst = edge_index[1]
    npad = E_PAD - E
    src_p = jnp.concatenate([src, jnp.zeros((npad,), jnp.int32)])
    dst_p = jnp.concatenate([dst, jnp.full((npad,), N, jnp.int32)])

    zeros_big = jnp.zeros((N_PAD, HID), jnp.float32)
    ones_c = jnp.ones((C, HID), jnp.float32)

    _sc_degree, _sc_spmm = _sc_kernels()
    deg2 = _sc_degree(dst_p, ones_c, zeros_big)
    mt0, dis = _tc_enc(deg2, x, W_enc.T, b_enc[None], W0.T)

    acc0 = _sc_spmm(src_p, dst_p, mt0, zeros_big)
    h1, mt1 = _tc_layer(acc0, mt0, dis, mt0, b0[None], g0[None], be0[None],
                        W1.T, has_res=False)

    acc1 = _sc_spmm(src_p, dst_p, mt1, zeros_big)
    h2, mt2 = _tc_layer(acc1, mt1, dis, h1, b1[None], g1[None], be1[None],
                        W2.T, has_res=True)

    acc2 = _sc_spmm(src_p, dst_p, mt2, zeros_big)
    sums, cnt = _tc_final(acc2, mt2, dis, h2, b2[None], g2[None], be2[None],
                          batch[:, None])

    return _tc_mlp(sums, cnt, Wm1.T, bm1[None], Wm2.T, bm2[None], Wm3.T,
                   bm3[None])


# speedup vs baseline: 14.4814x; 1.7044x over previous
import functools

import jax
import jax.numpy as jnp
import numpy as np
from jax import lax
from jax.experimental import pallas as pl
from jax.experimental.pallas import tpu as pltpu
from jax.experimental.pallas import tpu_sc as plsc

N = 10000
E = 320000
HID = 128
NG = 64
FFN = 300
EPS = 1e-5

NC = 2
NS = 16
NW = NC * NS
C = 128
CPW = 79
E_PAD = NW * CPW * C
TOT = 2 * CPW
CPW0 = 107
CPW1 = TOT - CPW0
CPWMAX = max(CPW0, CPW1)
RPT = 632
N_PAD = RPT * NS

DW = 16
BN = 1000
GRID = N // BN


def _sc_degree_body(dst_hbm, ones_hbm, zeros_hbm, out_hbm, dstv, ones_v, acc):
    cid = lax.axis_index("c")
    sid = lax.axis_index("s")
    w = sid * NC + cid
    pltpu.sync_copy(zeros_hbm.at[pl.ds(sid * RPT, RPT)],
                    acc.at[pl.ds(sid * RPT, RPT)])
    pltpu.sync_copy(dst_hbm.at[pl.ds(w * CPW * C, CPW * C)], dstv)
    pltpu.sync_copy(ones_hbm, ones_v)
    plsc.subcore_barrier()

    def body(ci, carry):
        pltpu.sync_copy(ones_v, acc.at[dstv.at[pl.ds(ci * C, C)]], add=True)
        return carry

    lax.fori_loop(0, CPW, body, 0)
    plsc.subcore_barrier()
    pltpu.sync_copy(acc.at[pl.ds(sid * RPT, RPT)],
                    out_hbm.at[cid, pl.ds(sid * RPT, RPT)])


def _sc_spmm_body(src_hbm, dst_hbm, mt_hbm, zeros_hbm, out_hbm,
                  srcv, ixd0, ixd1, rows0, rows1, acc,
                  g0, g1, d0, d1):
    cid = lax.axis_index("c")
    sid = lax.axis_index("s")
    pltpu.sync_copy(zeros_hbm.at[pl.ds(sid * RPT, RPT)],
                    acc.at[pl.ds(sid * RPT, RPT)])

    def pipe(start, n):
        base = start * C
        pltpu.sync_copy(src_hbm.at[pl.ds(base, n * C)],
                        srcv.at[pl.ds(0, n * C)])

        def gather(c, buf, sem):
            pltpu.async_copy(mt_hbm.at[srcv.at[pl.ds(c * C, C)]], buf, sem)

        def gwait(c, buf, sem):
            pltpu.make_async_copy(mt_hbm.at[srcv.at[pl.ds(c * C, C)]], buf,
                                  sem).wait()

        def dload(c, buf, sem):
            pltpu.async_copy(dst_hbm.at[pl.ds(base + c * C, C)], buf, sem)

        def dwait(c, buf, sem):
            pltpu.make_async_copy(dst_hbm.at[pl.ds(base + c * C, C)], buf,
                                  sem).wait()

        def scat(buf, ixd):
            pltpu.sync_copy(buf, acc.at[ixd], add=True)

        dload(0, ixd0, d0)
        gather(0, rows0, g0)
        dload(1, ixd1, d1)

        def body(i, carry):
            c0 = 2 * i
            gather(c0 + 1, rows1, g1)
            gwait(c0, rows0, g0)
            dwait(c0, ixd0, d0)
            scat(rows0, ixd0)
            dload(c0 + 2, ixd0, d0)
            gather(c0 + 2, rows0, g0)
            gwait(c0 + 1, rows1, g1)
            dwait(c0 + 1, ixd1, d1)
            scat(rows1, ixd1)
            dload(c0 + 3, ixd1, d1)
            return carry

        lax.fori_loop(0, (n - 1) // 2, body, 0)
        gwait(n - 1, rows0, g0)
        dwait(n - 1, ixd0, d0)
        scat(rows0, ixd0)
        dwait(n, ixd1, d1)

    plsc.subcore_barrier()

    @pl.when(cid == 0)
    def _():
        pipe(sid * TOT, CPW0)

    @pl.when(cid == 1)
    def _():
        pipe(sid * TOT + CPW0, CPW1)

    plsc.subcore_barrier()
    pltpu.sync_copy(acc.at[pl.ds(sid * RPT, RPT)],
                    out_hbm.at[cid, pl.ds(sid * RPT, RPT)])


@functools.lru_cache(maxsize=None)
def _sc_kernels():
    mesh = plsc.VectorSubcoreMesh(
        core_axis_name="c", subcore_axis_name="s",
        num_cores=NC, num_subcores=NS)
    sc_degree = pl.kernel(
        _sc_degree_body,
        out_type=jax.ShapeDtypeStruct((NC, N_PAD, HID), jnp.float32),
        mesh=mesh,
        scratch_types=[
            pltpu.VMEM((CPW * C,), jnp.int32),
            pltpu.VMEM((C, HID), jnp.float32),
            pltpu.VMEM_SHARED((N_PAD, HID), jnp.float32),
        ],
    )
    sc_spmm = pl.kernel(
        _sc_spmm_body,
        out_type=jax.ShapeDtypeStruct((NC, N_PAD, HID), jnp.float32),
        mesh=mesh,
        scratch_types=[
            pltpu.VMEM((CPWMAX * C,), jnp.int32),
            pltpu.VMEM((C,), jnp.int32),
            pltpu.VMEM((C,), jnp.int32),
            pltpu.VMEM((C, HID), jnp.float32),
            pltpu.VMEM((C, HID), jnp.float32),
            pltpu.VMEM_SHARED((N_PAD, HID), jnp.float32),
            pltpu.SemaphoreType.DMA,
            pltpu.SemaphoreType.DMA,
            pltpu.SemaphoreType.DMA,
            pltpu.SemaphoreType.DMA,
        ],
    )
    return sc_degree, sc_spmm


_ISQ = float(1.0 / np.sqrt(np.float32(1.0) + np.float32(EPS)))


def _enc_body(deg_ref, x_ref, wencT_ref, benc_ref, w0T_ref, mt0_ref, dis_ref):
    deg = deg_ref[0][:, :1] + deg_ref[1][:, :1] + 1.0
    dis = lax.rsqrt(deg)
    h = jnp.dot(x_ref[...], wencT_ref[...],
                preferred_element_type=jnp.float32) + benc_ref[...]
    mt0_ref[...] = jnp.dot(h, w0T_ref[...],
                           preferred_element_type=jnp.float32) * dis
    dis_ref[...] = dis


def _layer_body(has_res, acc_ref, mt_ref, dis_ref, hprev_ref, b_ref, g_ref,
                be_ref, wnT_ref, h_ref, mtn_ref):
    dis = dis_ref[...]
    agg = (acc_ref[0] + acc_ref[1] + mt_ref[...]) * dis
    h2 = (agg + b_ref[...]) * _ISQ * g_ref[...] + be_ref[...]
    h = jnp.maximum(h2, 0.0)
    if has_res:
        h = h + hprev_ref[...]
    h_ref[...] = h
    mtn_ref[...] = jnp.dot(h, wnT_ref[...],
                           preferred_element_type=jnp.float32) * dis


def _final_body(acc_ref, mt_ref, dis_ref, hprev_ref, b_ref, g_ref, be_ref,
                batch_ref, sums_ref, cnt_ref):
    i = pl.program_id(0)
    agg = (acc_ref[0] + acc_ref[1] + mt_ref[...]) * dis_ref[...]
    h2 = (agg + b_ref[...]) * _ISQ * g_ref[...] + be_ref[...]
    h = jnp.maximum(h2, 0.0) + hprev_ref[...]
    onehot = (batch_ref[...] ==
              lax.broadcasted_iota(jnp.int32, (1, NG), 1)).astype(jnp.float32)
    ps = lax.dot_general(onehot, h, (((0,), (0,)), ((), ())),
                         preferred_element_type=jnp.float32,
                         precision=lax.Precision.HIGHEST)
    pc = lax.dot_general(onehot, jnp.ones((BN, 1), jnp.float32),
                         (((0,), (0,)), ((), ())),
                         preferred_element_type=jnp.float32,
                         precision=lax.Precision.HIGHEST)

    @pl.when(i == 0)
    def _():
        sums_ref[...] = ps
        cnt_ref[...] = pc

    @pl.when(i > 0)
    def _():
        sums_ref[...] += ps
        cnt_ref[...] += pc


def _mlp_body(sums_ref, cnt_ref, w1T_ref, b1_ref, w2T_ref, b2_ref, w3T_ref,
              b3_ref, o_ref):
    gm = sums_ref[...] / jnp.maximum(cnt_ref[...], 1.0)
    o = jnp.maximum(jnp.dot(gm, w1T_ref[...],
                            preferred_element_type=jnp.float32) + b1_ref[...],
                    0.0)
    o = jnp.maximum(jnp.dot(o, w2T_ref[...],
                            preferred_element_type=jnp.float32) + b2_ref[...],
                    0.0)
    o_ref[...] = jnp.dot(o, w3T_ref[...],
                         preferred_element_type=jnp.float32) + b3_ref[...]


def _row_spec(cols):
    return pl.BlockSpec((BN, cols), lambda i: (i, 0))


_ACC_SPEC_1 = pl.BlockSpec((NC, BN, HID), lambda i: (0, i, 0))
_ACC_SPEC_H = pl.BlockSpec((NC, BN, HID), lambda i: (0, i, 0))
_FULL = pl.BlockSpec(index_map=lambda i: (0, 0))


def _tc_enc(deg2, x, wencT, benc, w0T):
    return pl.pallas_call(
        _enc_body,
        grid=(GRID,),
        in_specs=[_ACC_SPEC_1, _row_spec(HID), _FULL, _FULL, _FULL],
        out_specs=[_row_spec(HID), _row_spec(1)],
        out_shape=[jax.ShapeDtypeStruct((N, HID), jnp.float32),
                   jax.ShapeDtypeStruct((N, 1), jnp.float32)],
    )(deg2, x, wencT, benc, w0T)


def _tc_layer(acc2, mt, dis, hprev, b, g, be, wnT, has_res):
    return pl.pallas_call(
        functools.partial(_layer_body, has_res),
        grid=(GRID,),
        in_specs=[_ACC_SPEC_H, _row_spec(HID), _row_spec(1), _row_spec(HID),
                  _FULL, _FULL, _FULL, _FULL],
        out_specs=[_row_spec(HID), _row_spec(HID)],
        out_shape=[jax.ShapeDtypeStruct((N, HID), jnp.float32),
                   jax.ShapeDtypeStruct((N, HID), jnp.float32)],
    )(acc2, mt, dis, hprev, b, g, be, wnT)


def _tc_final(acc2, mt, dis, hprev, b, g, be, batch2d):
    return pl.pallas_call(
        _final_body,
        grid=(GRID,),
        in_specs=[_ACC_SPEC_H, _row_spec(HID), _row_spec(1), _row_spec(HID),
                  _FULL, _FULL, _FULL, _row_spec(1)],
        out_specs=[pl.BlockSpec((NG, HID), lambda i: (0, 0)),
                   pl.BlockSpec((NG, 1), lambda i: (0, 0))],
        out_shape=[jax.ShapeDtypeStruct((NG, HID), jnp.float32),
                   jax.ShapeDtypeStruct((NG, 1), jnp.float32)],
    )(acc2, mt, dis, hprev, b, g, be, batch2d)


def _tc_mlp(sums, cnt, w1T, b1, w2T, b2, w3T, b3):
    return pl.pallas_call(
        _mlp_body,
        out_shape=jax.ShapeDtypeStruct((NG, 1), jnp.float32),
    )(sums, cnt, w1T, b1, w2T, b2, w3T, b3)


def kernel(x, edge_index, batch, W_enc, b_enc, W0, b0, g0, be0, W1, b1, g1,
           be1, W2, b2, g2, be2, Wm1, bm1, Wm2, bm2, Wm3, bm3):
    src = edge_index[0]
    dst = edge_index[1]
    npad = E_PAD - E
    src_p = jnp.concatenate([src, jnp.zeros((npad,), jnp.int32)])
    dst_p = jnp.concatenate([dst, jnp.full((npad + C,), N, jnp.int32)])

    zeros_big = jnp.zeros((N_PAD, HID), jnp.float32)
    ones_c = jnp.ones((C, HID), jnp.float32)

    _sc_degree, _sc_spmm = _sc_kernels()
    deg2 = _sc_degree(dst_p, ones_c, zeros_big)
    mt0, dis = _tc_enc(deg2, x, W_enc.T, b_enc[None], W0.T)

    acc0 = _sc_spmm(src_p, dst_p, mt0, zeros_big)
    h1, mt1 = _tc_layer(acc0, mt0, dis, mt0, b0[None], g0[None], be0[None],
                        W1.T, has_res=False)

    acc1 = _sc_spmm(src_p, dst_p, mt1, zeros_big)
    h2, mt2 = _tc_layer(acc1, mt1, dis, h1, b1[None], g1[None], be1[None],
                        W2.T, has_res=True)

    acc2 = _sc_spmm(src_p, dst_p, mt2, zeros_big)
    sums, cnt = _tc_final(acc2, mt2, dis, h2, b2[None], g2[None], be2[None],
                          batch[:, None])

    return _tc_mlp(sums, cnt, Wm1.T, bm1[None], Wm2.T, bm2[None], Wm3.T,
                   bm3[None])
